# baseline (speedup 1.0000x reference)
import functools

import jax
import jax.numpy as jnp
from jax import lax
from jax.experimental import pallas as pl
from jax.experimental.pallas import tpu as pltpu
from jax.experimental.pallas import tpu_sc as plsc

_NODE_NUM = 100000
_N_UPD = 1600000

_NC = 2
_NS = 16
_NW = _NC * _NS

_LANE = 128
_NROWS = _N_UPD // _LANE
_ROWS_B = 384
_BIG = 26
_TAIL = _NROWS - (_ROWS_B * _NW + 8 * _BIG)
_TAIL_BASE = _NROWS - _TAIL
_RMAX = _ROWS_B + 8 + _TAIL

_NPAD = 100352
_SLICE = _NPAD // _NS
_PIPE = 16


def _scatter_body(idx_hbm, val_hbm, sums_hbm, counts_hbm,
                  idx_v, val_v, tmp_v, ones_v, sums_sh, counts_sh,
                  sem_a, sem_b):
  c = lax.axis_index("c")
  s = lax.axis_index("s")
  w = c * _NS + s

  def _zero(i, carry):
    tmp_v[pl.ds(i * 16, 16)] = jnp.zeros((16,), jnp.float32)
    return carry
  lax.fori_loop(0, _SLICE // 16, _zero, 0)
  pltpu.sync_copy(tmp_v, sums_sh.at[pl.ds(s * _SLICE, _SLICE)])
  pltpu.sync_copy(tmp_v, counts_sh.at[pl.ds(s * _SLICE, _SLICE)])

  for i in range(_LANE // 16):
    ones_v[pl.ds(i * 16, 16)] = jnp.ones((16,), jnp.float32)

  base = pl.multiple_of(w * _ROWS_B + 8 * jnp.minimum(w, _BIG), 8)
  pltpu.sync_copy(idx_hbm.at[pl.ds(base, _ROWS_B)], idx_v.at[pl.ds(0, _ROWS_B)])
  pltpu.sync_copy(val_hbm.at[pl.ds(base, _ROWS_B)], val_v.at[pl.ds(0, _ROWS_B)])

  @pl.when(w < _BIG)
  def _():
    off = pl.multiple_of(base + _ROWS_B, 8)
    pltpu.sync_copy(idx_hbm.at[pl.ds(off, 8)], idx_v.at[pl.ds(_ROWS_B, 8)])
    pltpu.sync_copy(val_hbm.at[pl.ds(off, 8)], val_v.at[pl.ds(_ROWS_B, 8)])

  @pl.when(w == 0)
  def _():
    pltpu.sync_copy(idx_hbm.at[pl.ds(_TAIL_BASE, _TAIL)],
                    idx_v.at[pl.ds(_ROWS_B + 8, _TAIL)])
    pltpu.sync_copy(val_hbm.at[pl.ds(_TAIL_BASE, _TAIL)],
                    val_v.at[pl.ds(_ROWS_B + 8, _TAIL)])

  nrows = (_ROWS_B + 8 * jnp.where(w < _BIG, 1, 0)
           + _TAIL * jnp.where(w == 0, 1, 0))

  plsc.subcore_barrier()

  def _fire(j):
    pltpu.async_copy(val_v.at[j], sums_sh.at[idx_v.at[j]], sem_a, add=True)
    pltpu.async_copy(ones_v, counts_sh.at[idx_v.at[j]], sem_b, add=True)

  def _drain_row():
    pltpu.make_async_copy(val_v.at[0], sums_sh.at[pl.ds(0, _LANE)],
                          sem_a).wait()
    pltpu.make_async_copy(ones_v, counts_sh.at[pl.ds(0, _LANE)],
                          sem_b).wait()

  def _prime(j, carry):
    _fire(j)
    return carry
  lax.fori_loop(0, _PIPE, _prime, 0)

  def _steady(j, carry):
    _fire(j)
    _drain_row()
    return carry
  lax.fori_loop(_PIPE, nrows, _steady, 0)

  def _tail(i, carry):
    _drain_row()
    return carry
  lax.fori_loop(0, _PIPE, _tail, 0)

  plsc.subcore_barrier()

  obase = c * _NPAD + s * _SLICE
  pltpu.sync_copy(sums_sh.at[pl.ds(s * _SLICE, _SLICE)],
                  sums_hbm.at[pl.ds(obase, _SLICE)])
  pltpu.sync_copy(counts_sh.at[pl.ds(s * _SLICE, _SLICE)],
                  counts_hbm.at[pl.ds(obase, _SLICE)])


_scatter_kernel = functools.partial(
    pl.kernel,
    out_type=(jax.ShapeDtypeStruct((_NC * _NPAD,), jnp.float32),
              jax.ShapeDtypeStruct((_NC * _NPAD,), jnp.float32)),
    mesh=plsc.VectorSubcoreMesh(core_axis_name="c", subcore_axis_name="s",
                                num_cores=_NC, num_subcores=_NS),
    scratch_types=(
        pltpu.VMEM((_RMAX, _LANE), jnp.int32),
        pltpu.VMEM((_RMAX, _LANE), jnp.float32),
        pltpu.VMEM((_SLICE,), jnp.float32),
        pltpu.VMEM((_LANE,), jnp.float32),
        pltpu.VMEM_SHARED((_NPAD,), jnp.float32),
        pltpu.VMEM_SHARED((_NPAD,), jnp.float32),
        pltpu.SemaphoreType.DMA,
        pltpu.SemaphoreType.DMA,
    ),
)(_scatter_body)


def _combine_body(state_ref, s_ref, c_ref, o_ref):
  sums = s_ref[pl.ds(0, _NODE_NUM)] + s_ref[pl.ds(_NPAD, _NODE_NUM)]
  counts = c_ref[pl.ds(0, _NODE_NUM)] + c_ref[pl.ds(_NPAD, _NODE_NUM)]
  o_ref[...] = state_ref[...] + sums / jnp.maximum(counts, 1.0)


def _combine(state, sums, counts):
  return pl.pallas_call(
      _combine_body,
      out_shape=jax.ShapeDtypeStruct((_NODE_NUM,), jnp.float32),
  )(state, sums, counts)


def kernel(node_errors_state, node_errors, node_indices):
  idx2d = node_indices.reshape(_NROWS, _LANE)
  val2d = node_errors.reshape(_NROWS, _LANE)
  sums, counts = _scatter_kernel(idx2d, val2d)
  return _combine(node_errors_state, sums, counts)


# speedup vs baseline: 1.0973x; 1.0973x over previous
import functools

import jax
import jax.numpy as jnp
from jax import lax
from jax.experimental import pallas as pl
from jax.experimental.pallas import tpu as pltpu
from jax.experimental.pallas import tpu_sc as plsc

_NODE_NUM = 100000
_N_UPD = 1600000

_NC = 2
_NS = 16
_NW = _NC * _NS

_LANE = 128
_NROWS = _N_UPD // _LANE
_ROWS_B = 384
_BIG = 26
_TAIL = _NROWS - (_ROWS_B * _NW + 8 * _BIG)
_TAIL_BASE = _NROWS - _TAIL
_RMAX = _ROWS_B + 8 + _TAIL

_NPAD = 100352
_SLICE = _NPAD // _NS
_PIPE = 8


def _scatter_body(idx_hbm, val_hbm, sums_hbm, counts_hbm,
                  idx_v, val_v, tmp_v, ones_v, sums_sh, counts_sh,
                  sem_a, sem_b):
  c = lax.axis_index("c")
  s = lax.axis_index("s")
  w = c * _NS + s

  base = pl.multiple_of(w * _ROWS_B + 8 * jnp.minimum(w, _BIG), 8)
  pltpu.async_copy(idx_hbm.at[pl.ds(base, _ROWS_B)],
                   idx_v.at[pl.ds(0, _ROWS_B)], sem_a)
  pltpu.async_copy(val_hbm.at[pl.ds(base, _ROWS_B)],
                   val_v.at[pl.ds(0, _ROWS_B)], sem_a)

  @pl.when(w < _BIG)
  def _():
    off = pl.multiple_of(base + _ROWS_B, 8)
    pltpu.async_copy(idx_hbm.at[pl.ds(off, 8)], idx_v.at[pl.ds(_ROWS_B, 8)],
                     sem_b)
    pltpu.async_copy(val_hbm.at[pl.ds(off, 8)], val_v.at[pl.ds(_ROWS_B, 8)],
                     sem_b)

  @pl.when(w == 0)
  def _():
    pltpu.async_copy(idx_hbm.at[pl.ds(_TAIL_BASE, _TAIL)],
                     idx_v.at[pl.ds(_ROWS_B + 8, _TAIL)], sem_b)
    pltpu.async_copy(val_hbm.at[pl.ds(_TAIL_BASE, _TAIL)],
                     val_v.at[pl.ds(_ROWS_B + 8, _TAIL)], sem_b)

  def _zero(i, carry):
    tmp_v[pl.ds(i * 16, 16)] = jnp.zeros((16,), jnp.float32)
    return carry
  lax.fori_loop(0, _SLICE // 16, _zero, 0)
  pltpu.sync_copy(tmp_v, sums_sh.at[pl.ds(s * _SLICE, _SLICE)])
  pltpu.sync_copy(tmp_v, counts_sh.at[pl.ds(s * _SLICE, _SLICE)])

  for i in range(_LANE // 16):
    ones_v[pl.ds(i * 16, 16)] = jnp.ones((16,), jnp.float32)

  nrows = (_ROWS_B + 8 * jnp.where(w < _BIG, 1, 0)
           + _TAIL * jnp.where(w == 0, 1, 0))

  pltpu.make_async_copy(idx_hbm.at[pl.ds(0, _ROWS_B)],
                        idx_v.at[pl.ds(0, _ROWS_B)], sem_a).wait()
  pltpu.make_async_copy(val_hbm.at[pl.ds(0, _ROWS_B)],
                        val_v.at[pl.ds(0, _ROWS_B)], sem_a).wait()

  @pl.when(w < _BIG)
  def _():
    pltpu.make_async_copy(idx_hbm.at[pl.ds(0, 8)],
                          idx_v.at[pl.ds(_ROWS_B, 8)], sem_b).wait()
    pltpu.make_async_copy(val_hbm.at[pl.ds(0, 8)],
                          val_v.at[pl.ds(_ROWS_B, 8)], sem_b).wait()

  @pl.when(w == 0)
  def _():
    pltpu.make_async_copy(idx_hbm.at[pl.ds(0, _TAIL)],
                          idx_v.at[pl.ds(_ROWS_B + 8, _TAIL)], sem_b).wait()
    pltpu.make_async_copy(val_hbm.at[pl.ds(0, _TAIL)],
                          val_v.at[pl.ds(_ROWS_B + 8, _TAIL)], sem_b).wait()

  plsc.subcore_barrier()

  def _fire(j):
    pltpu.async_copy(val_v.at[j], sums_sh.at[idx_v.at[j]], sem_a, add=True)
    pltpu.async_copy(ones_v, counts_sh.at[idx_v.at[j]], sem_b, add=True)

  def _drain_row():
    pltpu.make_async_copy(val_v.at[0], sums_sh.at[pl.ds(0, _LANE)],
                          sem_a).wait()
    pltpu.make_async_copy(ones_v, counts_sh.at[pl.ds(0, _LANE)],
                          sem_b).wait()

  def _prime(j, carry):
    _fire(j)
    return carry
  lax.fori_loop(0, _PIPE, _prime, 0)

  def _steady(j, carry):
    _fire(j)
    _drain_row()
    return carry
  lax.fori_loop(_PIPE, nrows, _steady, 0)

  def _tail(i, carry):
    _drain_row()
    return carry
  lax.fori_loop(0, _PIPE, _tail, 0)

  plsc.subcore_barrier()

  obase = c * _NPAD + s * _SLICE
  oa = pltpu.async_copy(sums_sh.at[pl.ds(s * _SLICE, _SLICE)],
                        sums_hbm.at[pl.ds(obase, _SLICE)], sem_a)
  ob = pltpu.async_copy(counts_sh.at[pl.ds(s * _SLICE, _SLICE)],
                        counts_hbm.at[pl.ds(obase, _SLICE)], sem_b)
  oa.wait()
  ob.wait()


_scatter_kernel = functools.partial(
    pl.kernel,
    out_type=(jax.ShapeDtypeStruct((_NC * _NPAD,), jnp.float32),
              jax.ShapeDtypeStruct((_NC * _NPAD,), jnp.float32)),
    mesh=plsc.VectorSubcoreMesh(core_axis_name="c", subcore_axis_name="s",
                                num_cores=_NC, num_subcores=_NS),
    scratch_types=(
        pltpu.VMEM((_RMAX, _LANE), jnp.int32),
        pltpu.VMEM((_RMAX, _LANE), jnp.float32),
        pltpu.VMEM((_SLICE,), jnp.float32),
        pltpu.VMEM((_LANE,), jnp.float32),
        pltpu.VMEM_SHARED((_NPAD,), jnp.float32),
        pltpu.VMEM_SHARED((_NPAD,), jnp.float32),
        pltpu.SemaphoreType.DMA,
        pltpu.SemaphoreType.DMA,
    ),
)(_scatter_body)


def _combine_body(state_ref, s_ref, c_ref, o_ref):
  sums = s_ref[pl.ds(0, _NODE_NUM)] + s_ref[pl.ds(_NPAD, _NODE_NUM)]
  counts = c_ref[pl.ds(0, _NODE_NUM)] + c_ref[pl.ds(_NPAD, _NODE_NUM)]
  o_ref[...] = state_ref[...] + sums / jnp.maximum(counts, 1.0)


def _combine(state, sums, counts):
  return pl.pallas_call(
      _combine_body,
      out_shape=jax.ShapeDtypeStruct((_NODE_NUM,), jnp.float32),
  )(state, sums, counts)


def kernel(node_errors_state, node_errors, node_indices):
  idx2d = node_indices.reshape(_NROWS, _LANE)
  val2d = node_errors.reshape(_NROWS, _LANE)
  sums, counts = _scatter_kernel(idx2d, val2d)
  return _combine(node_errors_state, sums, counts)
